# baseline (speedup 1.0000x reference)
import functools

import jax
import jax.numpy as jnp
from jax.experimental import pallas as pl
from jax.experimental.pallas import tpu as pltpu

N = 10000
D = 512
CH = 200
NC = N // CH
NBUF = 4
XCH = 1000
NXC = N // XCH


def _body(x_hbm, w_ref, bias_ref, adj_hbm, out_hbm,
          sup_ref, xbuf, abuf, obuf, xsem, asem, osem):
    def acp(c, b):
        return pltpu.make_async_copy(
            adj_hbm.at[pl.ds(c * CH, CH), :], abuf.at[b], asem.at[b])

    def xcp(c, b):
        return pltpu.make_async_copy(
            x_hbm.at[pl.ds(c * XCH, XCH), :], xbuf.at[b], xsem.at[b])

    def ocp(c, b):
        return pltpu.make_async_copy(
            obuf.at[b], out_hbm.at[pl.ds(c * CH, CH), :], osem.at[b])

    for b in range(NBUF):
        acp(b, b).start()
    xcp(0, 0).start()
    xcp(1, 1).start()

    w_bf = w_ref[...]
    for j in range(NXC):
        xcp(j, j % 2).wait()
        sup_ref[pl.ds(j * XCH, XCH), :] = jnp.dot(
            xbuf[j % 2].astype(jnp.bfloat16), w_bf,
            preferred_element_type=jnp.float32,
        ).astype(jnp.bfloat16)
        if j + 2 < NXC:
            xcp(j + 2, j % 2).start()

    sup = sup_ref[...]
    bias_v = bias_ref[...]

    def step(i, _):
        b = jax.lax.rem(i, NBUF)
        ob = jax.lax.rem(i, 2)
        pltpu.make_async_copy(
            adj_hbm.at[pl.ds(i * CH, CH), :], abuf.at[b], asem.at[b]).wait()
        res = jnp.dot(
            abuf[b].astype(jnp.bfloat16), sup,
            preferred_element_type=jnp.float32,
        ) + bias_v

        @pl.when(i >= 2)
        def _():
            pltpu.make_async_copy(
                obuf.at[ob], out_hbm.at[pl.ds((i - 2) * CH, CH), :],
                osem.at[ob]).wait()

        obuf[ob] = res
        pltpu.make_async_copy(
            obuf.at[ob], out_hbm.at[pl.ds(i * CH, CH), :], osem.at[ob]).start()

        @pl.when(i + NBUF < NC)
        def _():
            pltpu.make_async_copy(
                adj_hbm.at[pl.ds((i + NBUF) * CH, CH), :], abuf.at[b],
                asem.at[b]).start()

        return 0

    jax.lax.fori_loop(0, NC, step, 0)

    ocp(NC - 2, 0).wait()
    ocp(NC - 1, 1).wait()


@functools.partial(jax.jit, static_argnames=())
def kernel(adj, input, weight, bias):
    w_bf = weight.astype(jnp.bfloat16)
    bias2d = bias.reshape(1, D)

    out = pl.pallas_call(
        _body,
        in_specs=[
            pl.BlockSpec(memory_space=pltpu.MemorySpace.HBM),
            pl.BlockSpec(memory_space=pltpu.MemorySpace.VMEM),
            pl.BlockSpec(memory_space=pltpu.MemorySpace.VMEM),
            pl.BlockSpec(memory_space=pltpu.MemorySpace.HBM),
        ],
        out_specs=pl.BlockSpec(memory_space=pltpu.MemorySpace.HBM),
        out_shape=jax.ShapeDtypeStruct((N, D), jnp.float32),
        scratch_shapes=[
            pltpu.VMEM((N, D), jnp.bfloat16),
            pltpu.VMEM((2, XCH, D), jnp.float32),
            pltpu.VMEM((NBUF, CH, N), jnp.float32),
            pltpu.VMEM((2, CH, D), jnp.float32),
            pltpu.SemaphoreType.DMA((2,)),
            pltpu.SemaphoreType.DMA((NBUF,)),
            pltpu.SemaphoreType.DMA((2,)),
        ],
    )(input, w_bf, bias2d, adj)

    return out


# speedup vs baseline: 1.0254x; 1.0254x over previous
import functools

import jax
import jax.numpy as jnp
from jax.experimental import pallas as pl
from jax.experimental.pallas import tpu as pltpu

N = 10000
D = 512
CH = 200
NC = N // CH
NBUF = 3
XCH = 1000
NXC = N // XCH


def _body(x_hbm, w_ref, bias_ref, adj_hbm, out_hbm,
          sup_ref, xbuf, abuf, obuf, xsem, asem, osem):
    def acp(c, b):
        return pltpu.make_async_copy(
            adj_hbm.at[pl.ds(c * CH, CH), :], abuf.at[b], asem.at[b])

    def xcp(c, b):
        return pltpu.make_async_copy(
            x_hbm.at[pl.ds(c * XCH, XCH), :], xbuf.at[b], xsem.at[b])

    def ocp(c, b):
        return pltpu.make_async_copy(
            obuf.at[b], out_hbm.at[pl.ds(c * CH, CH), :], osem.at[b])

    for b in range(NBUF):
        acp(b, b).start()
    xcp(0, 0).start()
    xcp(1, 1).start()

    w_bf = w_ref[...]
    for j in range(NXC):
        xcp(j, j % 2).wait()
        sup_ref[pl.ds(j * XCH, XCH), :] = jnp.dot(
            xbuf[j % 2].astype(jnp.bfloat16), w_bf,
            preferred_element_type=jnp.float32,
        ).astype(jnp.bfloat16)
        if j + 2 < NXC:
            xcp(j + 2, j % 2).start()

    sup = sup_ref[...]
    bias_v = bias_ref[...]

    def step(i, _):
        b = jax.lax.rem(i, NBUF)
        ob = jax.lax.rem(i, 2)
        pltpu.make_async_copy(
            adj_hbm.at[pl.ds(i * CH, CH), :], abuf.at[b], asem.at[b]).wait()
        res = jnp.dot(
            abuf[b].astype(jnp.bfloat16), sup,
            preferred_element_type=jnp.float32,
        ) + bias_v

        @pl.when(i >= 2)
        def _():
            pltpu.make_async_copy(
                obuf.at[ob], out_hbm.at[pl.ds((i - 2) * CH, CH), :],
                osem.at[ob]).wait()

        obuf[ob] = res
        pltpu.make_async_copy(
            obuf.at[ob], out_hbm.at[pl.ds(i * CH, CH), :], osem.at[ob]).start()

        @pl.when(i + NBUF < NC)
        def _():
            pltpu.make_async_copy(
                adj_hbm.at[pl.ds((i + NBUF) * CH, CH), :], abuf.at[b],
                asem.at[b]).start()

        return 0

    jax.lax.fori_loop(0, NC, step, 0)

    ocp(NC - 2, 0).wait()
    ocp(NC - 1, 1).wait()


@functools.partial(jax.jit, static_argnames=())
def kernel(adj, input, weight, bias):
    w_bf = weight.astype(jnp.bfloat16)
    bias2d = bias.reshape(1, D)

    out = pl.pallas_call(
        _body,
        in_specs=[
            pl.BlockSpec(memory_space=pltpu.MemorySpace.HBM),
            pl.BlockSpec(memory_space=pltpu.MemorySpace.VMEM),
            pl.BlockSpec(memory_space=pltpu.MemorySpace.VMEM),
            pl.BlockSpec(memory_space=pltpu.MemorySpace.HBM),
        ],
        out_specs=pl.BlockSpec(memory_space=pltpu.MemorySpace.HBM),
        out_shape=jax.ShapeDtypeStruct((N, D), jnp.float32),
        scratch_shapes=[
            pltpu.VMEM((N, D), jnp.bfloat16),
            pltpu.VMEM((2, XCH, D), jnp.float32),
            pltpu.VMEM((NBUF, CH, N), jnp.float32),
            pltpu.VMEM((2, CH, D), jnp.float32),
            pltpu.SemaphoreType.DMA((2,)),
            pltpu.SemaphoreType.DMA((NBUF,)),
            pltpu.SemaphoreType.DMA((2,)),
        ],
    )(input, w_bf, bias2d, adj)

    return out


# speedup vs baseline: 1.0870x; 1.0600x over previous
import functools

import jax
import jax.numpy as jnp
from jax.experimental import pallas as pl
from jax.experimental.pallas import tpu as pltpu

N = 10000
D = 512
CH = 200
NC = N // CH
NBUF = 3
XCH = 2000
NXC = N // XCH


def _body(x_hbm, w_ref, bias_ref, adj_hbm, out_hbm,
          sup_ref, xbuf, abuf, obuf, xsem, asem, osem):
    def acp(c, b):
        return pltpu.make_async_copy(
            adj_hbm.at[pl.ds(c * CH, CH), :], abuf.at[b], asem.at[b])

    def xcp(c, b):
        return pltpu.make_async_copy(
            x_hbm.at[pl.ds(c * XCH, XCH), :], xbuf.at[b], xsem.at[b])

    def ocp(c, b):
        return pltpu.make_async_copy(
            obuf.at[b], out_hbm.at[pl.ds(c * CH, CH), :], osem.at[b])

    for b in range(NBUF):
        acp(b, b).start()
    xcp(0, 0).start()
    xcp(1, 1).start()

    w_v = w_ref[...]
    for j in range(NXC):
        xcp(j, j % 2).wait()
        sup_ref[pl.ds(j * XCH, XCH), :] = jnp.dot(
            xbuf[j % 2], w_v,
            precision=jax.lax.Precision.DEFAULT,
            preferred_element_type=jnp.float32,
        )
        if j + 2 < NXC:
            xcp(j + 2, j % 2).start()

    sup = sup_ref[...]
    bias_v = bias_ref[...]

    def step(i, _):
        b = jax.lax.rem(i, NBUF)
        ob = jax.lax.rem(i, 2)
        pltpu.make_async_copy(
            adj_hbm.at[pl.ds(i * CH, CH), :], abuf.at[b], asem.at[b]).wait()
        res = jnp.dot(
            abuf[b], sup,
            precision=jax.lax.Precision.DEFAULT,
            preferred_element_type=jnp.float32,
        ) + bias_v

        @pl.when(i >= 2)
        def _():
            pltpu.make_async_copy(
                obuf.at[ob], out_hbm.at[pl.ds((i - 2) * CH, CH), :],
                osem.at[ob]).wait()

        obuf[ob] = res
        pltpu.make_async_copy(
            obuf.at[ob], out_hbm.at[pl.ds(i * CH, CH), :], osem.at[ob]).start()

        @pl.when(i + NBUF < NC)
        def _():
            pltpu.make_async_copy(
                adj_hbm.at[pl.ds((i + NBUF) * CH, CH), :], abuf.at[b],
                asem.at[b]).start()

        return 0

    jax.lax.fori_loop(0, NC, step, 0)

    ocp(NC - 2, 0).wait()
    ocp(NC - 1, 1).wait()


@functools.partial(jax.jit, static_argnames=())
def kernel(adj, input, weight, bias):
    bias2d = bias.reshape(1, D)

    out = pl.pallas_call(
        _body,
        in_specs=[
            pl.BlockSpec(memory_space=pltpu.MemorySpace.HBM),
            pl.BlockSpec(memory_space=pltpu.MemorySpace.VMEM),
            pl.BlockSpec(memory_space=pltpu.MemorySpace.VMEM),
            pl.BlockSpec(memory_space=pltpu.MemorySpace.HBM),
        ],
        out_specs=pl.BlockSpec(memory_space=pltpu.MemorySpace.HBM),
        out_shape=jax.ShapeDtypeStruct((N, D), jnp.float32),
        scratch_shapes=[
            pltpu.VMEM((N, D), jnp.float32),
            pltpu.VMEM((2, XCH, D), jnp.float32),
            pltpu.VMEM((NBUF, CH, N), jnp.float32),
            pltpu.VMEM((2, CH, D), jnp.float32),
            pltpu.SemaphoreType.DMA((2,)),
            pltpu.SemaphoreType.DMA((NBUF,)),
            pltpu.SemaphoreType.DMA((2,)),
        ],
    )(input, weight, bias2d, adj)

    return out
